# baseline (speedup 1.0000x reference)
import functools

import jax
import jax.numpy as jnp
import numpy as np
from jax import lax
from jax.experimental import pallas as pl
from jax.experimental.pallas import tpu as pltpu
from jax.experimental.pallas import tpu_sc as plsc

_MAX_DEG = 64
_B = 16384
_NUM_SAMPLES = 32
_NC = 2
_NS = 16
_NW = _NC * _NS
_BPW = _B // _NW
_VPAD = 100016
_BLK = 16
_BLK_LAG = 16

_COLS_STATIC = np.array(
    [35, 45, 31, 63, 7, 4, 29, 44, 16, 58, 37, 19, 61, 2, 34, 5,
     30, 42, 3, 39, 56, 22, 6, 54, 18, 10, 11, 53, 32, 15, 49, 50],
    dtype=np.int32)


def _fixed_cols() -> np.ndarray:
    try:
        with jax.default_device(jax.devices("cpu")[0]):
            perm = jax.random.permutation(jax.random.key(42), _MAX_DEG)
            return np.asarray(perm, dtype=np.int32)[:_NUM_SAMPLES]
    except Exception:
        return _COLS_STATIC


_COLS = _fixed_cols()


def _make_sampler():
    mesh = plsc.VectorSubcoreMesh(core_axis_name="c", subcore_axis_name="s")

    @functools.partial(
        pl.kernel,
        mesh=mesh,
        out_type=jax.ShapeDtypeStruct((_B * _NUM_SAMPLES,), jnp.int32),
        scratch_types=[
            pltpu.VMEM((_BPW,), jnp.int32),
            pltpu.VMEM((_BPW, _MAX_DEG), jnp.int32),
            pltpu.VMEM((_BPW * _NUM_SAMPLES,), jnp.int32),
            pltpu.VMEM((_NUM_SAMPLES,), jnp.int32),
            pltpu.SemaphoreType.DMA,
        ],
        compiler_params=pltpu.CompilerParams(
            disable_bounds_checks=True,
            disable_semaphore_checks=True,
            skip_device_barrier=True,
        ),
    )
    def sampler(adj_hbm, ids_hbm, cols_hbm, out_hbm, idx_v, rows_v, out_v,
                cols_v, sem):
        wid = lax.axis_index("s") * _NC + lax.axis_index("c")
        base = wid * _BPW
        pltpu.sync_copy(cols_hbm, cols_v)
        pltpu.sync_copy(ids_hbm.at[pl.ds(base, _BPW)], idx_v)

        c0 = cols_v[pl.ds(0, 16)]
        c1 = cols_v[pl.ds(16, 16)]
        w0 = c0 & 15
        w1 = c1 & 15
        q0 = c0 >> 4
        q1 = c1 >> 4

        def vgather(v, w):
            return v.at[w].get(mode="promise_in_bounds")

        def permute(v0, v1, v2, v3, w, q):
            r01 = jnp.where(q == 0, vgather(v0, w), vgather(v1, w))
            r23 = jnp.where(q == 2, vgather(v2, w), vgather(v3, w))
            return jnp.where(q < 2, r01, r23)

        def fire_block(b):
            idvec = jnp.full((16,), _VPAD - 1, jnp.int32) - idx_v[
                pl.ds(b * _BLK, 16)]
            for k in range(_BLK):
                pltpu.async_copy(adj_hbm.at[idvec[k]],
                                 rows_v.at[b * _BLK + k], sem)

        def drain_block(b):
            pltpu.make_async_copy(adj_hbm.at[pl.ds(0, _BLK)],
                                  rows_v.at[pl.ds(b * _BLK, _BLK)], sem).wait()

        def permute_block(b):
            for k in range(_BLK):
                r = b * _BLK + k
                v0 = rows_v[r, pl.ds(0, 16)]
                v1 = rows_v[r, pl.ds(16, 16)]
                v2 = rows_v[r, pl.ds(32, 16)]
                v3 = rows_v[r, pl.ds(48, 16)]
                out_v[pl.ds(r * _NUM_SAMPLES, 16)] = (
                    permute(v0, v1, v2, v3, w0, q0))
                out_v[pl.ds(r * _NUM_SAMPLES + 16, 16)] = (
                    permute(v0, v1, v2, v3, w1, q1))

        def step(b, carry):
            fire_block(b)

            @pl.when(b >= _BLK_LAG)
            def _():
                drain_block(b - _BLK_LAG)
                permute_block(b - _BLK_LAG)

            return carry

        nblk = _BPW // _BLK
        lax.fori_loop(0, nblk, step, 0, unroll=False)

        def tail(b, carry):
            drain_block(b)
            permute_block(b)
            return carry

        lax.fori_loop(nblk - _BLK_LAG, nblk, tail, 0, unroll=False)

        pltpu.sync_copy(out_v,
                        out_hbm.at[pl.ds(base * _NUM_SAMPLES,
                                         _BPW * _NUM_SAMPLES)])

    return sampler


_sampler = _make_sampler()


def kernel(adj_info, ids, num_samples):
    del num_samples
    rev = jnp.arange(_VPAD - 1, -1, -1, dtype=jnp.int32)
    adj_rev = jnp.take(adj_info, rev, axis=0, mode="clip")
    flat = _sampler(adj_rev, ids, jnp.asarray(_COLS))
    return flat.reshape(_B, _NUM_SAMPLES)


# speedup vs baseline: 1.7593x; 1.7593x over previous
import functools

import jax
import jax.numpy as jnp
import numpy as np
from jax import lax
from jax.experimental import pallas as pl
from jax.experimental.pallas import tpu as pltpu
from jax.experimental.pallas import tpu_sc as plsc

_MAX_DEG = 64
_B = 16384
_NUM_SAMPLES = 32
_NC = 2
_NS = 16
_NW = _NC * _NS
_BPW = _B // _NW
_BLK = 16
_BLK_LAG = 16

_COLS_STATIC = np.array(
    [35, 45, 31, 63, 7, 4, 29, 44, 16, 58, 37, 19, 61, 2, 34, 5,
     30, 42, 3, 39, 56, 22, 6, 54, 18, 10, 11, 53, 32, 15, 49, 50],
    dtype=np.int32)


def _fixed_cols() -> np.ndarray:
    try:
        with jax.default_device(jax.devices("cpu")[0]):
            perm = jax.random.permutation(jax.random.key(42), _MAX_DEG)
            return np.asarray(perm, dtype=np.int32)[:_NUM_SAMPLES]
    except Exception:
        return _COLS_STATIC


_COLS = _fixed_cols()


def _make_sampler():
    mesh = plsc.VectorSubcoreMesh(core_axis_name="c", subcore_axis_name="s")

    @functools.partial(
        pl.kernel,
        mesh=mesh,
        out_type=jax.ShapeDtypeStruct((_B * _NUM_SAMPLES,), jnp.int32),
        scratch_types=[
            pltpu.VMEM((_BPW,), jnp.int32),
            pltpu.VMEM((_BPW, _MAX_DEG), jnp.int32),
            pltpu.VMEM((_BPW * _NUM_SAMPLES,), jnp.int32),
            pltpu.VMEM((_NUM_SAMPLES,), jnp.int32),
            pltpu.SemaphoreType.DMA,
        ],
        compiler_params=pltpu.CompilerParams(
            disable_bounds_checks=True,
            disable_semaphore_checks=True,
            skip_device_barrier=True,
        ),
    )
    def sampler(adj_hbm, ids_hbm, cols_hbm, out_hbm, idx_v, rows_v, out_v,
                cols_v, sem):
        wid = lax.axis_index("s") * _NC + lax.axis_index("c")
        base = wid * _BPW
        pltpu.sync_copy(cols_hbm, cols_v)
        pltpu.sync_copy(ids_hbm.at[pl.ds(base, _BPW)], idx_v)

        c0 = cols_v[pl.ds(0, 16)]
        c1 = cols_v[pl.ds(16, 16)]
        w0 = c0 & 15
        w1 = c1 & 15
        q0 = c0 >> 4
        q1 = c1 >> 4

        def vgather(v, w):
            return v.at[w].get(mode="promise_in_bounds")

        def permute(v0, v1, v2, v3, w, q):
            r01 = jnp.where(q == 0, vgather(v0, w), vgather(v1, w))
            r23 = jnp.where(q == 2, vgather(v2, w), vgather(v3, w))
            return jnp.where(q < 2, r01, r23)

        def fire_block(b):
            idvec = idx_v[pl.ds(b * _BLK, 16)]
            for k in range(_BLK):
                pltpu.async_copy(adj_hbm.at[idvec[k]],
                                 rows_v.at[b * _BLK + k], sem)

        def drain_block(b):
            pltpu.make_async_copy(adj_hbm.at[pl.ds(0, _BLK)],
                                  rows_v.at[pl.ds(b * _BLK, _BLK)], sem).wait()

        def permute_block(b):
            for k in range(_BLK):
                r = b * _BLK + k
                v0 = rows_v[r, pl.ds(0, 16)]
                v1 = rows_v[r, pl.ds(16, 16)]
                v2 = rows_v[r, pl.ds(32, 16)]
                v3 = rows_v[r, pl.ds(48, 16)]
                out_v[pl.ds(r * _NUM_SAMPLES, 16)] = (
                    permute(v0, v1, v2, v3, w0, q0))
                out_v[pl.ds(r * _NUM_SAMPLES + 16, 16)] = (
                    permute(v0, v1, v2, v3, w1, q1))

        def step(b, carry):
            fire_block(b)

            @pl.when(b >= _BLK_LAG)
            def _():
                drain_block(b - _BLK_LAG)
                permute_block(b - _BLK_LAG)

            return carry

        nblk = _BPW // _BLK
        lax.fori_loop(0, nblk, step, 0, unroll=False)

        def tail(b, carry):
            drain_block(b)
            permute_block(b)
            return carry

        lax.fori_loop(nblk - _BLK_LAG, nblk, tail, 0, unroll=False)

        pltpu.sync_copy(out_v,
                        out_hbm.at[pl.ds(base * _NUM_SAMPLES,
                                         _BPW * _NUM_SAMPLES)])

    return sampler


_sampler = _make_sampler()


def kernel(adj_info, ids, num_samples):
    del num_samples
    flat = _sampler(adj_info, ids, jnp.asarray(_COLS))
    return flat.reshape(_B, _NUM_SAMPLES)
